# baseline (speedup 1.0000x reference)
import functools

import jax
import jax.numpy as jnp
from jax import lax
from jax.experimental import pallas as pl
from jax.experimental.pallas import tpu as pltpu
from jax.experimental.pallas import tpu_sc as plsc

_EPS = 1e-5
_NSC = 2
_NTILE = 16


def _sc_gather_pair(xr, xc, row, col):
    e = row.shape[0]
    nw = _NSC * _NTILE
    eperw = e // nw
    chunk = 200
    idxg = 5
    assert eperw % (chunk * idxg) == 0
    nchunks = eperw // chunk

    mesh = plsc.VectorSubcoreMesh(core_axis_name="c", subcore_axis_name="s")

    @functools.partial(
        pl.kernel,
        out_type=jax.ShapeDtypeStruct((e, 128), jnp.bfloat16),
        mesh=mesh,
        scratch_types=[
            pltpu.VMEM((chunk * idxg,), jnp.int32),
            pltpu.VMEM((chunk * idxg,), jnp.int32),
            pltpu.VMEM((chunk, 128), jnp.bfloat16),
            pltpu.VMEM((chunk, 128), jnp.bfloat16),
            pltpu.SemaphoreType.DMA,
            pltpu.SemaphoreType.DMA,
        ],
        compiler_params=pltpu.CompilerParams(use_tc_tiling_on_sc=False),
    )
    def gather_kernel(xr_hbm, xc_hbm, row_hbm, col_hbm, out,
                      ir, ic, br, bc, semr, semc):
        c = lax.axis_index("c")
        s = lax.axis_index("s")
        w = s * _NSC + c

        def body(k, carry):
            base = w * eperw + k * chunk

            @pl.when(k % idxg == 0)
            def _load_idx():
                pltpu.sync_copy(row_hbm.at[pl.ds(base, chunk * idxg)], ir)
                pltpu.sync_copy(col_hbm.at[pl.ds(base, chunk * idxg)], ic)

            o = (k % idxg) * chunk
            cr = pltpu.async_copy(xr_hbm.at[ir.at[pl.ds(o, chunk)]], br,
                                  semr)
            cc = pltpu.async_copy(xc_hbm.at[ic.at[pl.ds(o, chunk)]], bc,
                                  semc)
            cr.wait()
            cc.wait()

            def add_row(i, carry2):
                for j in range(4):
                    sl = pl.ds(j * 32, 32)
                    br[i, sl] += bc[i, sl]
                return carry2

            lax.fori_loop(0, chunk, add_row, 0)
            pltpu.sync_copy(br, out.at[pl.ds(base, chunk)])
            return carry

        lax.fori_loop(0, nchunks, body, 0)

    return gather_kernel(xr, xc, row, col)


def _sc_segment_sum(msg2, col_a, col_b, num_nodes):
    ehalf = col_a.shape[0]
    npad = ((num_nodes + _NTILE * 8 - 1) // (_NTILE * 8)) * (_NTILE * 8)
    rows_per_tile = npad // _NTILE
    hpert = ehalf // _NTILE
    chunk = 1000
    assert hpert % chunk == 0
    nchunks = hpert // chunk

    zeros = jnp.zeros((rows_per_tile, 16), jnp.float32)
    mesh = plsc.VectorSubcoreMesh(core_axis_name="c", subcore_axis_name="s")
    out_t = jax.ShapeDtypeStruct((npad, 16), jnp.float32)

    @functools.partial(
        pl.kernel,
        out_type=[out_t, out_t, out_t, out_t],
        mesh=mesh,
        scratch_types=[
            pltpu.VMEM((chunk,), jnp.int32),
            pltpu.VMEM((chunk, 16), jnp.float32),
            pltpu.VMEM_SHARED((npad, 16), jnp.float32),
        ],
        compiler_params=pltpu.CompilerParams(use_tc_tiling_on_sc=False),
    )
    def scatter_kernel(msg_hbm, ce_hbm, co_hbm, z_hbm,
                       o0, o1, o2, o3, idx_v, upd_v, acc):
        c = lax.axis_index("c")
        s = lax.axis_index("s")
        r0 = s * rows_per_tile

        def run(qq, out_hbm):
            pltpu.sync_copy(z_hbm, acc.at[pl.ds(r0, rows_per_tile)])
            plsc.subcore_barrier()

            def body(k, carry):
                base = s * hpert + k * chunk
                pltpu.sync_copy(ce_hbm.at[pl.ds(base, chunk)], idx_v)
                pltpu.sync_copy(
                    msg_hbm.at[pl.ds(base, chunk), pl.ds(16 * qq, 16)],
                    upd_v)
                pltpu.sync_copy(upd_v, acc.at[idx_v], add=True)
                pltpu.sync_copy(co_hbm.at[pl.ds(base, chunk)], idx_v)
                pltpu.sync_copy(
                    msg_hbm.at[pl.ds(base, chunk), pl.ds(64 + 16 * qq, 16)],
                    upd_v)
                pltpu.sync_copy(upd_v, acc.at[idx_v], add=True)
                return carry

            lax.fori_loop(0, nchunks, body, 0)
            plsc.subcore_barrier()
            pltpu.sync_copy(acc.at[pl.ds(r0, rows_per_tile)],
                            out_hbm.at[pl.ds(r0, rows_per_tile)])
            plsc.subcore_barrier()

        @pl.when(c == 0)
        def _lo():
            run(0, o0)
            run(1, o1)

        @pl.when(c == 1)
        def _hi():
            run(2, o2)
            run(3, o3)

    outs = scatter_kernel(msg2, col_a, col_b, zeros)
    return [o[:num_nodes] for o in outs]


def _softplus(x):
    return jnp.maximum(x, 0.0) + jnp.log1p(jnp.exp(-jnp.abs(x)))


def _edge_body(nedges, gpair, nbr, wn, bfc, g1, b1, msg2, stats):
    p = pl.program_id(0)
    j = pl.program_id(1)

    @pl.when((p == 0) & (j == 0))
    def _init():
        stats[...] = jnp.zeros_like(stats)

    gated = (
        gpair[...].astype(jnp.float32)
        + jnp.dot(nbr[...], wn[...], preferred_element_type=jnp.float32)
        + bfc[...]
    )

    @pl.when(p == 0)
    def _accum():
        stats[0:1, :] += jnp.sum(gated, axis=0, keepdims=True)
        stats[1:2, :] += jnp.sum(gated * gated, axis=0, keepdims=True)

    @pl.when(p == 1)
    def _apply():
        @pl.when(j == 0)
        def _finalize():
            mean = stats[0:1, :] / nedges
            var = stats[1:2, :] / nedges - mean * mean
            scale = g1[...] / jnp.sqrt(var + _EPS)
            stats[2:3, :] = scale
            stats[3:4, :] = b1[...] - mean * scale

        z = gated * stats[2:3, :] + stats[3:4, :]
        half = z.shape[1] // 2
        msg = jax.nn.sigmoid(z[:, :half]) * _softplus(z[:, half:])
        h = msg.shape[0] // 2
        msg2[...] = jnp.concatenate([msg[:h], msg[h:]], axis=1)


def _edge_messages(gpair, nbr_fea, wn, bfc, g1, b1):
    e = gpair.shape[0]
    nbrl = nbr_fea.shape[1]
    width = gpair.shape[1]
    eblk = 4000 if e % 4000 == 0 else e
    nb = e // eblk

    grid = (2, nb)

    out = pl.pallas_call(
        functools.partial(_edge_body, float(e)),
        grid=grid,
        in_specs=[
            pl.BlockSpec((eblk, width), lambda p, j: (j, 0)),
            pl.BlockSpec((eblk, nbrl), lambda p, j: (j, 0)),
            pl.BlockSpec((nbrl, width), lambda p, j: (0, 0)),
            pl.BlockSpec((1, width), lambda p, j: (0, 0)),
            pl.BlockSpec((1, width), lambda p, j: (0, 0)),
            pl.BlockSpec((1, width), lambda p, j: (0, 0)),
        ],
        out_specs=pl.BlockSpec((eblk // 2, width),
                               lambda p, j: (jnp.where(p == 1, j, 0), 0)),
        out_shape=jax.ShapeDtypeStruct((e // 2, width), jnp.float32),
        scratch_shapes=[pltpu.VMEM((8, width), jnp.float32)],
    )(gpair, nbr_fea, wn,
      bfc.reshape(1, -1), g1.reshape(1, -1), b1.reshape(1, -1))
    return out


def kernel(atom_fea, nbr_fea, nbr_fea_idx, dists, crystal_atom_idx, batch,
           W_emb, b_emb, Wfc, bfc, g1, b1, g2, b2, W_c2f, b_c2f, W_out, b_out):
    n = atom_fea.shape[0]
    ncrys = 256
    afl = W_emb.shape[1]

    x = atom_fea @ W_emb + b_emb
    row = nbr_fea_idx[0]
    col = nbr_fea_idx[1]
    e = col.shape[0]
    eblk = 4000 if e % 4000 == 0 else e
    col_blocks = col.reshape(-1, 2, eblk // 2)
    col_a = col_blocks[:, 0, :].reshape(-1)
    col_b = col_blocks[:, 1, :].reshape(-1)

    for i in range(Wfc.shape[0]):
        xr = x @ Wfc[i][:afl]
        xc = x @ Wfc[i][afl:2 * afl]
        mr = jnp.mean(xr, axis=0)
        mc = jnp.mean(xc, axis=0)
        xr = (xr - mr).astype(jnp.bfloat16)
        xc = (xc - mc).astype(jnp.bfloat16)
        gpair = _sc_gather_pair(xr, xc, row, col)
        msg2 = _edge_messages(gpair, nbr_fea, Wfc[i][2 * afl:],
                              bfc[i] + mr + mc, g1[i], b1[i])
        sums_q = _sc_segment_sum(msg2, col_a, col_b, n)
        summed = jnp.concatenate(sums_q, axis=1)
        m = jnp.mean(summed, axis=0, keepdims=True)
        v = jnp.var(summed, axis=0, keepdims=True)
        summed = g2[i] * (summed - m) / jnp.sqrt(v + _EPS) + b2[i]
        x = _softplus(x + summed)

    sums = jax.ops.segment_sum(x, batch, num_segments=ncrys)
    counts = jax.ops.segment_sum(jnp.ones((n, 1), x.dtype), batch,
                                 num_segments=ncrys)
    crys = sums / jnp.maximum(counts, 1.0)
    crys = _softplus(crys) @ W_c2f + b_c2f
    crys = _softplus(crys)
    return crys @ W_out + b_out


# speedup vs baseline: 1.2616x; 1.2616x over previous
import functools

import jax
import jax.numpy as jnp
from jax import lax
from jax.experimental import pallas as pl
from jax.experimental.pallas import tpu as pltpu
from jax.experimental.pallas import tpu_sc as plsc

_EPS = 1e-5
_NSC = 2
_NTILE = 16


def _sc_gather_pair(xr, xc, row, col):
    e = row.shape[0]
    nw = _NSC * _NTILE
    eperw = e // nw
    chunk = 200
    idxg = 5
    assert eperw % (chunk * idxg) == 0
    nchunks = eperw // chunk

    mesh = plsc.VectorSubcoreMesh(core_axis_name="c", subcore_axis_name="s")

    @functools.partial(
        pl.kernel,
        out_type=jax.ShapeDtypeStruct((e, 128), jnp.float32),
        mesh=mesh,
        scratch_types=[
            pltpu.VMEM((chunk * idxg,), jnp.int32),
            pltpu.VMEM((chunk * idxg,), jnp.int32),
            pltpu.VMEM((chunk, 128), jnp.float32),
            pltpu.VMEM((chunk, 128), jnp.float32),
            pltpu.SemaphoreType.DMA,
            pltpu.SemaphoreType.DMA,
        ],
        compiler_params=pltpu.CompilerParams(use_tc_tiling_on_sc=False),
    )
    def gather_kernel(xr_hbm, xc_hbm, row_hbm, col_hbm, out,
                      ir, ic, br, bc, semr, semc):
        c = lax.axis_index("c")
        s = lax.axis_index("s")
        w = s * _NSC + c

        def body(k, carry):
            base = w * eperw + k * chunk

            @pl.when(k % idxg == 0)
            def _load_idx():
                pltpu.sync_copy(row_hbm.at[pl.ds(base, chunk * idxg)], ir)
                pltpu.sync_copy(col_hbm.at[pl.ds(base, chunk * idxg)], ic)

            o = (k % idxg) * chunk
            cr = pltpu.async_copy(xr_hbm.at[ir.at[pl.ds(o, chunk)]], br,
                                  semr)
            cc = pltpu.async_copy(xc_hbm.at[ic.at[pl.ds(o, chunk)]], bc,
                                  semc)
            cr.wait()
            cc.wait()

            def add_row(i, carry2):
                for j in range(8):
                    sl = pl.ds(j * 16, 16)
                    br[i, sl] += bc[i, sl]
                return carry2

            lax.fori_loop(0, chunk, add_row, 0)
            pltpu.sync_copy(br, out.at[pl.ds(base, chunk)])
            return carry

        lax.fori_loop(0, nchunks, body, 0)

    return gather_kernel(xr, xc, row, col)


def _sc_segment_sum(msg2, col_a, col_b, num_nodes):
    ehalf = col_a.shape[0]
    npad = ((num_nodes + _NTILE * 8 - 1) // (_NTILE * 8)) * (_NTILE * 8)
    rows_per_tile = npad // _NTILE
    hpert = ehalf // _NTILE
    chunk = 1000
    assert hpert % chunk == 0
    nchunks = hpert // chunk

    zeros = jnp.zeros((rows_per_tile, 16), jnp.float32)
    mesh = plsc.VectorSubcoreMesh(core_axis_name="c", subcore_axis_name="s")
    out_t = jax.ShapeDtypeStruct((npad, 16), jnp.float32)

    @functools.partial(
        pl.kernel,
        out_type=[out_t, out_t, out_t, out_t],
        mesh=mesh,
        scratch_types=[
            pltpu.VMEM((chunk,), jnp.int32),
            pltpu.VMEM((chunk, 16), jnp.float32),
            pltpu.VMEM_SHARED((npad, 16), jnp.float32),
        ],
        compiler_params=pltpu.CompilerParams(use_tc_tiling_on_sc=False),
    )
    def scatter_kernel(msg_hbm, ce_hbm, co_hbm, z_hbm,
                       o0, o1, o2, o3, idx_v, upd_v, acc):
        c = lax.axis_index("c")
        s = lax.axis_index("s")
        r0 = s * rows_per_tile

        def run(qq, out_hbm):
            pltpu.sync_copy(z_hbm, acc.at[pl.ds(r0, rows_per_tile)])
            plsc.subcore_barrier()

            def body(k, carry):
                base = s * hpert + k * chunk
                pltpu.sync_copy(ce_hbm.at[pl.ds(base, chunk)], idx_v)
                pltpu.sync_copy(
                    msg_hbm.at[pl.ds(base, chunk), pl.ds(16 * qq, 16)],
                    upd_v)
                pltpu.sync_copy(upd_v, acc.at[idx_v], add=True)
                pltpu.sync_copy(co_hbm.at[pl.ds(base, chunk)], idx_v)
                pltpu.sync_copy(
                    msg_hbm.at[pl.ds(base, chunk), pl.ds(64 + 16 * qq, 16)],
                    upd_v)
                pltpu.sync_copy(upd_v, acc.at[idx_v], add=True)
                return carry

            lax.fori_loop(0, nchunks, body, 0)
            plsc.subcore_barrier()
            pltpu.sync_copy(acc.at[pl.ds(r0, rows_per_tile)],
                            out_hbm.at[pl.ds(r0, rows_per_tile)])
            plsc.subcore_barrier()

        @pl.when(c == 0)
        def _lo():
            run(0, o0)
            run(1, o1)

        @pl.when(c == 1)
        def _hi():
            run(2, o2)
            run(3, o3)

    outs = scatter_kernel(msg2, col_a, col_b, zeros)
    return [o[:num_nodes] for o in outs]


def _softplus(x):
    return jnp.maximum(x, 0.0) + jnp.log1p(jnp.exp(-jnp.abs(x)))


def _edge_body(nedges, gpair, nbr, wn, bfc, g1, b1, msg2, stats):
    p = pl.program_id(0)
    j = pl.program_id(1)

    @pl.when((p == 0) & (j == 0))
    def _init():
        stats[...] = jnp.zeros_like(stats)

    gated = (
        gpair[...].astype(jnp.float32)
        + jnp.dot(nbr[...], wn[...], preferred_element_type=jnp.float32)
        + bfc[...]
    )

    @pl.when(p == 0)
    def _accum():
        stats[0:1, :] += jnp.sum(gated, axis=0, keepdims=True)
        stats[1:2, :] += jnp.sum(gated * gated, axis=0, keepdims=True)

    @pl.when(p == 1)
    def _apply():
        @pl.when(j == 0)
        def _finalize():
            mean = stats[0:1, :] / nedges
            var = stats[1:2, :] / nedges - mean * mean
            scale = g1[...] / jnp.sqrt(var + _EPS)
            stats[2:3, :] = scale
            stats[3:4, :] = b1[...] - mean * scale

        z = gated * stats[2:3, :] + stats[3:4, :]
        half = z.shape[1] // 2
        msg = jax.nn.sigmoid(z[:, :half]) * _softplus(z[:, half:])
        h = msg.shape[0] // 2
        msg2[...] = jnp.concatenate([msg[:h], msg[h:]], axis=1)


def _edge_messages(gpair, nbr_fea, wn, bfc, g1, b1):
    e = gpair.shape[0]
    nbrl = nbr_fea.shape[1]
    width = gpair.shape[1]
    eblk = 4000 if e % 4000 == 0 else e
    nb = e // eblk

    grid = (2, nb)

    out = pl.pallas_call(
        functools.partial(_edge_body, float(e)),
        grid=grid,
        in_specs=[
            pl.BlockSpec((eblk, width), lambda p, j: (j, 0)),
            pl.BlockSpec((eblk, nbrl), lambda p, j: (j, 0)),
            pl.BlockSpec((nbrl, width), lambda p, j: (0, 0)),
            pl.BlockSpec((1, width), lambda p, j: (0, 0)),
            pl.BlockSpec((1, width), lambda p, j: (0, 0)),
            pl.BlockSpec((1, width), lambda p, j: (0, 0)),
        ],
        out_specs=pl.BlockSpec((eblk // 2, width),
                               lambda p, j: (jnp.where(p == 1, j, 0), 0)),
        out_shape=jax.ShapeDtypeStruct((e // 2, width), jnp.float32),
        scratch_shapes=[pltpu.VMEM((8, width), jnp.float32)],
    )(gpair, nbr_fea, wn,
      bfc.reshape(1, -1), g1.reshape(1, -1), b1.reshape(1, -1))
    return out


def kernel(atom_fea, nbr_fea, nbr_fea_idx, dists, crystal_atom_idx, batch,
           W_emb, b_emb, Wfc, bfc, g1, b1, g2, b2, W_c2f, b_c2f, W_out, b_out):
    n = atom_fea.shape[0]
    ncrys = 256
    afl = W_emb.shape[1]

    x = atom_fea @ W_emb + b_emb
    row = nbr_fea_idx[0]
    col = nbr_fea_idx[1]
    e = col.shape[0]
    eblk = 4000 if e % 4000 == 0 else e
    col_blocks = col.reshape(-1, 2, eblk // 2)
    col_a = col_blocks[:, 0, :].reshape(-1)
    col_b = col_blocks[:, 1, :].reshape(-1)

    for i in range(Wfc.shape[0]):
        xr = x @ Wfc[i][:afl]
        xc = x @ Wfc[i][afl:2 * afl]
        gpair = _sc_gather_pair(xr, xc, row, col)
        msg2 = _edge_messages(gpair, nbr_fea, Wfc[i][2 * afl:],
                              bfc[i], g1[i], b1[i])
        sums_q = _sc_segment_sum(msg2, col_a, col_b, n)
        summed = jnp.concatenate(sums_q, axis=1)
        m = jnp.mean(summed, axis=0, keepdims=True)
        v = jnp.var(summed, axis=0, keepdims=True)
        summed = g2[i] * (summed - m) / jnp.sqrt(v + _EPS) + b2[i]
        x = _softplus(x + summed)

    sums = jax.ops.segment_sum(x, batch, num_segments=ncrys)
    counts = jax.ops.segment_sum(jnp.ones((n, 1), x.dtype), batch,
                                 num_segments=ncrys)
    crys = sums / jnp.maximum(counts, 1.0)
    crys = _softplus(crys) @ W_c2f + b_c2f
    crys = _softplus(crys)
    return crys @ W_out + b_out
